# baseline (speedup 1.0000x reference)
import functools

import jax
import jax.numpy as jnp
from jax import lax
from jax.experimental import pallas as pl
from jax.experimental.pallas import tpu as pltpu
from jax.experimental.pallas import tpu_sc as plsc


@functools.lru_cache(maxsize=None)
def _build_sc_kernel(B, NCLS, D):
    info = plsc.get_sparse_core_info()
    NC, NS, L = info.num_cores, info.num_subcores, info.num_lanes
    NW = NC * NS
    BPW = B // NW
    R = 4
    NCHUNK = BPW // R
    U = 8
    GROUPS = D // L

    mesh = plsc.VectorSubcoreMesh(core_axis_name="c", subcore_axis_name="s")

    @functools.partial(
        pl.kernel,
        mesh=mesh,
        out_type=jax.ShapeDtypeStruct((B, D), jnp.float32),
        scratch_types=[
            pltpu.VMEM((BPW,), jnp.int32),
            pltpu.VMEM((NCLS, D), jnp.float32),
            pltpu.VMEM((NCLS, D), jnp.float32),
            pltpu.VMEM((R, D), jnp.float32),
            pltpu.VMEM((R, D), jnp.float32),
            pltpu.SemaphoreType.DMA,
            pltpu.SemaphoreType.DMA,
            pltpu.SemaphoreType.DMA,
            pltpu.SemaphoreType.DMA,
        ],
    )
    def sc_fma(lab_hbm, mean_hbm, std_hbm, noise_hbm, out_hbm,
               idx_v, mtab, stab, nbuf0, nbuf1,
               in_sem0, in_sem1, out_sem0, out_sem1):
        wid = lax.axis_index("s") * NC + lax.axis_index("c")
        base = wid * BPW
        nbufs = (nbuf0, nbuf1)
        in_sems, out_sems = (in_sem0, in_sem1), (out_sem0, out_sem1)

        pltpu.sync_copy(lab_hbm.at[wid], idx_v)
        h_m = pltpu.async_copy(mean_hbm, mtab, out_sem0)
        h_s = pltpu.async_copy(std_hbm, stab, out_sem1)

        def issue_in(c, p):
            pltpu.async_copy(noise_hbm.at[pl.ds(base + c * R, R)],
                             nbufs[p], in_sems[p])

        def wait_in(p):
            pltpu.make_async_copy(noise_hbm.at[pl.ds(base, R)], nbufs[p],
                                  in_sems[p]).wait()

        def issue_out(c, p):
            pltpu.async_copy(nbufs[p], out_hbm.at[pl.ds(base + c * R, R)],
                             out_sems[p])

        def wait_out(p):
            pltpu.make_async_copy(nbufs[p], out_hbm.at[pl.ds(base, R)],
                                  out_sems[p]).wait()

        def compute(group, quarter, p):
            nb = nbufs[p]
            labv = idx_v[pl.ds(pl.multiple_of(group * (4 * R), 16), L)]
            for r in range(R):
                lab = labv[quarter * R + r]

                def col_body(i, _, r=r, lab=lab):
                    for u in range(U):
                        off = (i * U + u) * L
                        n = nb[r, pl.ds(off, L)]
                        m = mtab[lab, pl.ds(off, L)]
                        s = stab[lab, pl.ds(off, L)]
                        nb[r, pl.ds(off, L)] = m + s * n
                    return 0

                lax.fori_loop(0, GROUPS // U, col_body, 0)

        issue_in(0, 0)
        h_m.wait()
        h_s.wait()

        issue_in(1, 1)
        wait_in(0)
        compute(0, 0, 0)
        issue_out(0, 0)

        def quad(i, _):
            for k in (1, 2, 3, 4):
                c = 4 * i + k
                quarter = k % 4
                group = i + (1 if k == 4 else 0)
                p = k % 2
                wait_out(1 - p)
                issue_in(c + 1, 1 - p)
                wait_in(p)
                compute(group, quarter, p)
                issue_out(c, p)
            return 0

        lax.fori_loop(0, (NCHUNK - 4) // 4, quad, 0)

        for c, quarter, p in ((NCHUNK - 3, 1, 1), (NCHUNK - 2, 2, 0)):
            wait_out(1 - p)
            issue_in(c + 1, 1 - p)
            wait_in(p)
            compute(c // 4, quarter, p)
            issue_out(c, p)

        wait_out(0)
        wait_in(1)
        compute((NCHUNK - 1) // 4, 3, 1)
        issue_out(NCHUNK - 1, 1)
        wait_out(1)

    return sc_fma, NW, BPW


def kernel(labels, class_means, class_stds, noise):
    B = labels.shape[0]
    NCLS = class_means.shape[0]
    D = class_means.shape[1] * class_means.shape[2] * class_means.shape[3]
    sc_fma, NW, BPW = _build_sc_kernel(B, NCLS, D)
    out = sc_fma(
        labels.astype(jnp.int32).reshape(NW, BPW),
        class_means.reshape(NCLS, D),
        class_stds.reshape(NCLS, D),
        noise.reshape(B, D),
    )
    return out.reshape(noise.shape)


# speedup vs baseline: 1.6359x; 1.6359x over previous
import functools

import jax
import jax.numpy as jnp
from jax import lax
from jax.experimental import pallas as pl
from jax.experimental.pallas import tpu as pltpu
from jax.experimental.pallas import tpu_sc as plsc


@functools.lru_cache(maxsize=None)
def _build_sc_kernel(B, NCLS, D):
    info = plsc.get_sparse_core_info()
    NC, NS, L = info.num_cores, info.num_subcores, info.num_lanes
    NW = NC * NS
    BPW = B // NW
    R = 4
    NCHUNK = BPW // R
    U = 8
    GROUPS = D // L

    mesh = plsc.VectorSubcoreMesh(core_axis_name="c", subcore_axis_name="s")

    @functools.partial(
        pl.kernel,
        mesh=mesh,
        out_type=jax.ShapeDtypeStruct((B, D), jnp.float32),
        scratch_types=[
            pltpu.VMEM((BPW,), jnp.int32),
            pltpu.VMEM((NCLS, D), jnp.float32),
            pltpu.VMEM((NCLS, D), jnp.float32),
            pltpu.VMEM((R, D), jnp.float32),
            pltpu.VMEM((R, D), jnp.float32),
            pltpu.SemaphoreType.DMA,
            pltpu.SemaphoreType.DMA,
            pltpu.SemaphoreType.DMA,
            pltpu.SemaphoreType.DMA,
        ],
    )
    def sc_fma(lab_hbm, mean_hbm, std_hbm, noise_hbm, out_hbm,
               idx_v, mtab, stab, nbuf0, nbuf1,
               in_sem0, in_sem1, out_sem0, out_sem1):
        wid = lax.axis_index("s") * NC + lax.axis_index("c")
        base = wid * BPW
        nbufs = (nbuf0, nbuf1)
        in_sems, out_sems = (in_sem0, in_sem1), (out_sem0, out_sem1)

        pltpu.sync_copy(lab_hbm.at[wid], idx_v)
        h_m = pltpu.async_copy(mean_hbm, mtab, out_sem0)
        h_s = pltpu.async_copy(std_hbm, stab, out_sem1)

        def issue_in(c, p):
            pltpu.async_copy(noise_hbm.at[pl.ds(base + c * R, R)],
                             nbufs[p], in_sems[p])

        def wait_in(p):
            pltpu.make_async_copy(noise_hbm.at[pl.ds(base, R)], nbufs[p],
                                  in_sems[p]).wait()

        def issue_out(c, p):
            pltpu.async_copy(nbufs[p], out_hbm.at[pl.ds(base + c * R, R)],
                             out_sems[p])

        def wait_out(p):
            pltpu.make_async_copy(nbufs[p], out_hbm.at[pl.ds(base, R)],
                                  out_sems[p]).wait()

        def compute(group, quarter, p):
            nb = nbufs[p]
            labv = idx_v[pl.ds(pl.multiple_of(group * (4 * R), 16), L)]
            for r in range(R):
                lab = labv[quarter * R + r]

                @plsc.parallel_loop(0, D, step=L, unroll=U)
                def col_body(off, r=r, lab=lab):
                    n = nb[r, pl.ds(off, L)]
                    m = mtab[lab, pl.ds(off, L)]
                    s = stab[lab, pl.ds(off, L)]
                    nb[r, pl.ds(off, L)] = m + s * n

        issue_in(0, 0)
        h_m.wait()
        h_s.wait()

        issue_in(1, 1)
        wait_in(0)
        compute(0, 0, 0)
        issue_out(0, 0)

        def quad(i, _):
            for k in (1, 2, 3, 4):
                c = 4 * i + k
                quarter = k % 4
                group = i + (1 if k == 4 else 0)
                p = k % 2
                wait_out(1 - p)
                issue_in(c + 1, 1 - p)
                wait_in(p)
                compute(group, quarter, p)
                issue_out(c, p)
            return 0

        lax.fori_loop(0, (NCHUNK - 4) // 4, quad, 0)

        for c, quarter, p in ((NCHUNK - 3, 1, 1), (NCHUNK - 2, 2, 0)):
            wait_out(1 - p)
            issue_in(c + 1, 1 - p)
            wait_in(p)
            compute(c // 4, quarter, p)
            issue_out(c, p)

        wait_out(0)
        wait_in(1)
        compute((NCHUNK - 1) // 4, 3, 1)
        issue_out(NCHUNK - 1, 1)
        wait_out(1)

    return sc_fma, NW, BPW


def kernel(labels, class_means, class_stds, noise):
    B = labels.shape[0]
    NCLS = class_means.shape[0]
    D = class_means.shape[1] * class_means.shape[2] * class_means.shape[3]
    sc_fma, NW, BPW = _build_sc_kernel(B, NCLS, D)
    out = sc_fma(
        labels.astype(jnp.int32).reshape(NW, BPW),
        class_means.reshape(NCLS, D),
        class_stds.reshape(NCLS, D),
        noise.reshape(B, D),
    )
    return out.reshape(noise.shape)
